# baseline (speedup 1.0000x reference)
import functools

import jax
import jax.numpy as jnp
from jax import lax
from jax.experimental import pallas as pl
from jax.experimental.pallas import tpu as pltpu
from jax.experimental.pallas import tpu_sc as plsc

N = 10000
E = 320000
C = 64
OC = 32
NCORES = 2
NSUB = 16
NT = NCORES * NSUB
EPT = E // NT
NPAD = 10240
EPAD = NPAD - N
B = 128
NCHUNK = (EPT + EPAD) // B
RPT = NPAD // NSUB
ZBLK = 128

_mesh = plsc.VectorSubcoreMesh(
    core_axis_name="c", subcore_axis_name="s",
    num_cores=NCORES, num_subcores=NSUB)


def _fill_f32(ref, rows, cols, value):
    def body(i, carry):
        for j in range(cols // 16):
            ref[i, pl.ds(j * 16, 16)] = jnp.full((16,), value, jnp.float32)
        return carry
    lax.fori_loop(0, rows, body, 0)


def _deg_body(er_hbm, out_hbm, dst_v, ones_v, zero_v, acc_sh):
    cid = lax.axis_index("c")
    sid = lax.axis_index("s")
    wid = cid * NSUB + sid
    _fill_f32(ones_v, B, 16, 1.0)
    _fill_f32(zero_v, ZBLK, 16, 0.0)
    for k in range(RPT // ZBLK):
        pltpu.sync_copy(zero_v, acc_sh.at[pl.ds(sid * RPT + k * ZBLK, ZBLK)])
    pltpu.sync_copy(er_hbm.at[1, wid], dst_v)
    plsc.subcore_barrier()

    def body(ci, carry):
        pltpu.sync_copy(ones_v, acc_sh.at[dst_v.at[ci]], add=True)
        return carry
    lax.fori_loop(0, NCHUNK, body, 0)

    plsc.subcore_barrier()
    pltpu.sync_copy(acc_sh.at[pl.ds(sid * RPT, RPT)],
                    out_hbm.at[cid, pl.ds(sid * RPT, RPT)])


def _make_deg_kernel(interpret=False):
    return functools.partial(
        pl.kernel,
        out_type=jax.ShapeDtypeStruct((NCORES, NPAD, 16), jnp.float32),
        mesh=_mesh,
        scratch_types=[
            pltpu.VMEM((NCHUNK, B), jnp.int32),
            pltpu.VMEM((B, 16), jnp.float32),
            pltpu.VMEM((ZBLK, 16), jnp.float32),
            pltpu.VMEM_SHARED((NPAD, 16), jnp.float32),
        ],
        compiler_params=pltpu.CompilerParams(use_tc_tiling_on_sc=False),
        interpret=interpret,
    )(_deg_body)


_deg_kernel = _make_deg_kernel()


def _agg_body(h_hbm, er_hbm, out_hbm, src_v, dst_v, rows0_v, rows1_v, zero_v,
              acc_sh, sem0, sem1):
    cid = lax.axis_index("c")
    sid = lax.axis_index("s")
    wid = cid * NSUB + sid
    _fill_f32(zero_v, ZBLK, C, 0.0)
    for k in range(RPT // ZBLK):
        pltpu.sync_copy(zero_v, acc_sh.at[pl.ds(sid * RPT + k * ZBLK, ZBLK)])
    pltpu.sync_copy(er_hbm.at[0, wid], src_v)
    pltpu.sync_copy(er_hbm.at[1, wid], dst_v)
    plsc.subcore_barrier()

    pltpu.async_copy(h_hbm.at[src_v.at[0]], rows0_v, sem0)

    @pl.loop(0, NCHUNK - 2, step=2)
    def _pair(ci):
        pltpu.async_copy(h_hbm.at[src_v.at[ci + 1]], rows1_v, sem1)
        pltpu.make_async_copy(h_hbm.at[src_v.at[ci]], rows0_v, sem0).wait()
        pltpu.sync_copy(rows0_v, acc_sh.at[dst_v.at[ci]], add=True)
        pltpu.async_copy(h_hbm.at[src_v.at[ci + 2]], rows0_v, sem0)
        pltpu.make_async_copy(h_hbm.at[src_v.at[ci + 1]], rows1_v, sem1).wait()
        pltpu.sync_copy(rows1_v, acc_sh.at[dst_v.at[ci + 1]], add=True)

    pltpu.async_copy(h_hbm.at[src_v.at[NCHUNK - 1]], rows1_v, sem1)
    pltpu.make_async_copy(h_hbm.at[src_v.at[NCHUNK - 2]], rows0_v, sem0).wait()
    pltpu.sync_copy(rows0_v, acc_sh.at[dst_v.at[NCHUNK - 2]], add=True)
    pltpu.make_async_copy(h_hbm.at[src_v.at[NCHUNK - 1]], rows1_v, sem1).wait()
    pltpu.sync_copy(rows1_v, acc_sh.at[dst_v.at[NCHUNK - 1]], add=True)

    plsc.subcore_barrier()
    pltpu.sync_copy(acc_sh.at[pl.ds(sid * RPT, RPT)],
                    out_hbm.at[cid, pl.ds(sid * RPT, RPT)])


def _make_agg_kernel(interpret=False):
    return functools.partial(
        pl.kernel,
        out_type=jax.ShapeDtypeStruct((NCORES, NPAD, C), jnp.float32),
        mesh=_mesh,
        scratch_types=[
            pltpu.VMEM((NCHUNK, B), jnp.int32),
            pltpu.VMEM((NCHUNK, B), jnp.int32),
            pltpu.VMEM((B, C), jnp.float32),
            pltpu.VMEM((B, C), jnp.float32),
            pltpu.VMEM((ZBLK, C), jnp.float32),
            pltpu.VMEM_SHARED((NPAD, C), jnp.float32),
            pltpu.SemaphoreType.DMA,
            pltpu.SemaphoreType.DMA,
        ],
        compiler_params=pltpu.CompilerParams(use_tc_tiling_on_sc=False),
        interpret=interpret,
    )(_agg_body)


_agg_kernel = _make_agg_kernel()


_BLK = 1000
_GRID = N // _BLK


def _dinv_block(degp):
    deg = degp[0, :, :1] + degp[1, :, :1] + 1.0
    return lax.rsqrt(deg)


def _h1p_body(x_ref, w_ref, degp_ref, o_ref):
    dinv = _dinv_block(degp_ref[...])
    h = jnp.dot(x_ref[...], w_ref[...],
                preferred_element_type=jnp.float32,
                precision=lax.Precision.HIGHEST)
    o_ref[...] = h * dinv


def _h1p_call(x, W1, degp):
    return pl.pallas_call(
        _h1p_body,
        grid=(_GRID,),
        in_specs=[
            pl.BlockSpec((_BLK, 128), lambda i: (i, 0)),
            pl.BlockSpec((128, C), lambda i: (0, 0)),
            pl.BlockSpec((NCORES, _BLK, 16), lambda i: (0, i, 0)),
        ],
        out_specs=pl.BlockSpec((_BLK, C), lambda i: (i, 0)),
        out_shape=jax.ShapeDtypeStruct((N, C), jnp.float32),
    )(x, W1, degp)


def _hp_body(raw_ref, h1p_ref, degp_ref, b_ref, o_ref):
    dinv = _dinv_block(degp_ref[...])
    raw = raw_ref[0] + raw_ref[1]
    h = jnp.maximum(dinv * (raw + h1p_ref[...]) + b_ref[...], 0.0)
    o_ref[...] = dinv * h


def _hp_call(raw1, h1p, degp, b1):
    return pl.pallas_call(
        _hp_body,
        grid=(_GRID,),
        in_specs=[
            pl.BlockSpec((NCORES, _BLK, C), lambda i: (0, i, 0)),
            pl.BlockSpec((_BLK, C), lambda i: (i, 0)),
            pl.BlockSpec((NCORES, _BLK, 16), lambda i: (0, i, 0)),
            pl.BlockSpec((1, C), lambda i: (0, 0)),
        ],
        out_specs=pl.BlockSpec((_BLK, C), lambda i: (i, 0)),
        out_shape=jax.ShapeDtypeStruct((N, C), jnp.float32),
    )(raw1, h1p, degp, b1)


def _out_body(raw_ref, hp_ref, degp_ref, wmu_ref, bmu_ref, wls_ref, bls_ref,
              mu_ref, ls_ref):
    dinv = _dinv_block(degp_ref[...])
    g = dinv * (raw_ref[0] + raw_ref[1] + hp_ref[...])
    mu_ref[...] = jnp.dot(g, wmu_ref[...],
                          preferred_element_type=jnp.float32,
                          precision=lax.Precision.HIGHEST) + bmu_ref[...]
    ls_ref[...] = jnp.dot(g, wls_ref[...],
                          preferred_element_type=jnp.float32,
                          precision=lax.Precision.HIGHEST) + bls_ref[...]


def _out_call(raw2, hp, degp, W_mu, b_mu, W_ls, b_ls):
    return pl.pallas_call(
        _out_body,
        grid=(_GRID,),
        in_specs=[
            pl.BlockSpec((NCORES, _BLK, C), lambda i: (0, i, 0)),
            pl.BlockSpec((_BLK, C), lambda i: (i, 0)),
            pl.BlockSpec((NCORES, _BLK, 16), lambda i: (0, i, 0)),
            pl.BlockSpec((C, OC), lambda i: (0, 0)),
            pl.BlockSpec((1, OC), lambda i: (0, 0)),
            pl.BlockSpec((C, OC), lambda i: (0, 0)),
            pl.BlockSpec((1, OC), lambda i: (0, 0)),
        ],
        out_specs=[
            pl.BlockSpec((_BLK, OC), lambda i: (i, 0)),
            pl.BlockSpec((_BLK, OC), lambda i: (i, 0)),
        ],
        out_shape=[
            jax.ShapeDtypeStruct((N, OC), jnp.float32),
            jax.ShapeDtypeStruct((N, OC), jnp.float32),
        ],
    )(raw2, hp, degp, W_mu, b_mu, W_ls, b_ls)


def kernel(x, W1, b1, W_mu, b_mu, W_ls, b_ls, edge_index):
    e2 = edge_index.astype(jnp.int32).reshape(2, NT, EPT)
    pad_dst = jnp.broadcast_to(N + jnp.arange(EPAD, dtype=jnp.int32),
                               (NT, EPAD))
    pad = jnp.stack([jnp.zeros((NT, EPAD), jnp.int32), pad_dst])
    er = jnp.concatenate([e2, pad], axis=2).reshape(2, NT, NCHUNK, B)
    degp = _deg_kernel(er)
    h1p = _h1p_call(x, W1, degp)
    raw1 = _agg_kernel(h1p, er)
    hp = _hp_call(raw1, h1p, degp, b1.reshape(1, C))
    raw2 = _agg_kernel(hp, er)
    mu, ls = _out_call(raw2, hp, degp, W_mu, b_mu.reshape(1, OC),
                       W_ls, b_ls.reshape(1, OC))
    return (mu, ls)


# speedup vs baseline: 1.8963x; 1.8963x over previous
import functools

import jax
import jax.numpy as jnp
from jax import lax
from jax.experimental import pallas as pl
from jax.experimental.pallas import tpu as pltpu
from jax.experimental.pallas import tpu_sc as plsc

N = 10000
E = 320000
C = 64
OC = 32
NCORES = 2
NSUB = 16
NT = NCORES * NSUB
EPT = E // NT
NPAD = 10240
B = 80
NCHUNK = EPT // B
RPT = NPAD // NSUB
ZBLK = 128

_mesh = plsc.VectorSubcoreMesh(
    core_axis_name="c", subcore_axis_name="s",
    num_cores=NCORES, num_subcores=NSUB)


def _fill_f32(ref, rows, cols, value):
    def body(i, carry):
        for j in range(cols // 16):
            ref[i, pl.ds(j * 16, 16)] = jnp.full((16,), value, jnp.float32)
        return carry
    lax.fori_loop(0, rows, body, 0)


def _deg_body(er_hbm, out_hbm, dst_v, ones_v, zero_v, acc_sh):
    cid = lax.axis_index("c")
    sid = lax.axis_index("s")
    wid = cid * NSUB + sid
    _fill_f32(ones_v, B, 16, 1.0)
    _fill_f32(zero_v, ZBLK, 16, 0.0)
    for k in range(RPT // ZBLK):
        pltpu.sync_copy(zero_v, acc_sh.at[pl.ds(sid * RPT + k * ZBLK, ZBLK)])
    pltpu.sync_copy(er_hbm.at[1, wid], dst_v)
    plsc.subcore_barrier()

    def body(ci, carry):
        pltpu.sync_copy(ones_v, acc_sh.at[dst_v.at[ci]], add=True)
        return carry
    lax.fori_loop(0, NCHUNK, body, 0)

    plsc.subcore_barrier()
    pltpu.sync_copy(acc_sh.at[pl.ds(sid * RPT, RPT)],
                    out_hbm.at[cid, pl.ds(sid * RPT, RPT)])


def _make_deg_kernel(interpret=False):
    return functools.partial(
        pl.kernel,
        out_type=jax.ShapeDtypeStruct((NCORES, NPAD, 16), jnp.float32),
        mesh=_mesh,
        scratch_types=[
            pltpu.VMEM((NCHUNK, B), jnp.int32),
            pltpu.VMEM((B, 16), jnp.float32),
            pltpu.VMEM((ZBLK, 16), jnp.float32),
            pltpu.VMEM_SHARED((NPAD, 16), jnp.float32),
        ],
        compiler_params=pltpu.CompilerParams(use_tc_tiling_on_sc=False),
        interpret=interpret,
    )(_deg_body)


_deg_kernel = _make_deg_kernel()


def _agg_body(h_hbm, er_hbm, out_hbm, src_v, dst_v, rows0_v, rows1_v, zero_v,
              acc_sh, sem0, sem1):
    cid = lax.axis_index("c")
    sid = lax.axis_index("s")
    wid = cid * NSUB + sid
    _fill_f32(zero_v, ZBLK, C, 0.0)
    for k in range(RPT // ZBLK):
        pltpu.sync_copy(zero_v, acc_sh.at[pl.ds(sid * RPT + k * ZBLK, ZBLK)])
    pltpu.sync_copy(er_hbm.at[0, wid], src_v)
    pltpu.sync_copy(er_hbm.at[1, wid], dst_v)
    plsc.subcore_barrier()

    pltpu.async_copy(h_hbm.at[src_v.at[0]], rows0_v, sem0)

    @pl.loop(0, NCHUNK - 1, step=2)
    def _pair(ci):
        pltpu.async_copy(h_hbm.at[src_v.at[ci + 1]], rows1_v, sem1)
        pltpu.make_async_copy(h_hbm.at[src_v.at[ci]], rows0_v, sem0).wait()
        pltpu.sync_copy(rows0_v, acc_sh.at[dst_v.at[ci]], add=True)
        pltpu.async_copy(h_hbm.at[src_v.at[ci + 2]], rows0_v, sem0)
        pltpu.make_async_copy(h_hbm.at[src_v.at[ci + 1]], rows1_v, sem1).wait()
        pltpu.sync_copy(rows1_v, acc_sh.at[dst_v.at[ci + 1]], add=True)

    pltpu.make_async_copy(h_hbm.at[src_v.at[NCHUNK - 1]], rows0_v, sem0).wait()
    pltpu.sync_copy(rows0_v, acc_sh.at[dst_v.at[NCHUNK - 1]], add=True)

    plsc.subcore_barrier()
    pltpu.sync_copy(acc_sh.at[pl.ds(sid * RPT, RPT)],
                    out_hbm.at[cid, pl.ds(sid * RPT, RPT)])


def _make_agg_kernel(interpret=False):
    return functools.partial(
        pl.kernel,
        out_type=jax.ShapeDtypeStruct((NCORES, NPAD, C), jnp.float32),
        mesh=_mesh,
        scratch_types=[
            pltpu.VMEM((NCHUNK, B), jnp.int32),
            pltpu.VMEM((NCHUNK, B), jnp.int32),
            pltpu.VMEM((B, C), jnp.float32),
            pltpu.VMEM((B, C), jnp.float32),
            pltpu.VMEM((ZBLK, C), jnp.float32),
            pltpu.VMEM_SHARED((NPAD, C), jnp.float32),
            pltpu.SemaphoreType.DMA,
            pltpu.SemaphoreType.DMA,
        ],
        compiler_params=pltpu.CompilerParams(use_tc_tiling_on_sc=False),
        interpret=interpret,
    )(_agg_body)


_agg_kernel = _make_agg_kernel()


_BLK = 1000
_GRID = N // _BLK


def _dinv_block(degp):
    deg = degp[0, :, :1] + degp[1, :, :1] + 1.0
    return lax.rsqrt(deg)


def _mm_body(x_ref, w_ref, o_ref):
    o_ref[...] = jnp.dot(x_ref[...], w_ref[...],
                         preferred_element_type=jnp.float32,
                         precision=lax.Precision.HIGHEST)


def _mm_call(x, W1):
    return pl.pallas_call(
        _mm_body,
        grid=(_GRID,),
        in_specs=[
            pl.BlockSpec((_BLK, 128), lambda i: (i, 0)),
            pl.BlockSpec((128, C), lambda i: (0, 0)),
        ],
        out_specs=pl.BlockSpec((_BLK, C), lambda i: (i, 0)),
        out_shape=jax.ShapeDtypeStruct((N, C), jnp.float32),
    )(x, W1)


def _scale_body(h_ref, degp_ref, o_ref):
    o_ref[...] = h_ref[...] * _dinv_block(degp_ref[...])


def _scale_call(h1, degp):
    return pl.pallas_call(
        _scale_body,
        grid=(_GRID,),
        in_specs=[
            pl.BlockSpec((_BLK, C), lambda i: (i, 0)),
            pl.BlockSpec((NCORES, _BLK, 16), lambda i: (0, i, 0)),
        ],
        out_specs=pl.BlockSpec((_BLK, C), lambda i: (i, 0)),
        out_shape=jax.ShapeDtypeStruct((N, C), jnp.float32),
    )(h1, degp)


def _hp_body(raw_ref, h1p_ref, degp_ref, b_ref, o_ref):
    dinv = _dinv_block(degp_ref[...])
    raw = raw_ref[0] + raw_ref[1]
    h = jnp.maximum(dinv * (raw + h1p_ref[...]) + b_ref[...], 0.0)
    o_ref[...] = dinv * h


def _hp_call(raw1, h1p, degp, b1):
    return pl.pallas_call(
        _hp_body,
        grid=(_GRID,),
        in_specs=[
            pl.BlockSpec((NCORES, _BLK, C), lambda i: (0, i, 0)),
            pl.BlockSpec((_BLK, C), lambda i: (i, 0)),
            pl.BlockSpec((NCORES, _BLK, 16), lambda i: (0, i, 0)),
            pl.BlockSpec((1, C), lambda i: (0, 0)),
        ],
        out_specs=pl.BlockSpec((_BLK, C), lambda i: (i, 0)),
        out_shape=jax.ShapeDtypeStruct((N, C), jnp.float32),
    )(raw1, h1p, degp, b1)


def _out_body(raw_ref, hp_ref, degp_ref, wmu_ref, bmu_ref, wls_ref, bls_ref,
              mu_ref, ls_ref):
    dinv = _dinv_block(degp_ref[...])
    g = dinv * (raw_ref[0] + raw_ref[1] + hp_ref[...])
    mu_ref[...] = jnp.dot(g, wmu_ref[...],
                          preferred_element_type=jnp.float32,
                          precision=lax.Precision.HIGHEST) + bmu_ref[...]
    ls_ref[...] = jnp.dot(g, wls_ref[...],
                          preferred_element_type=jnp.float32,
                          precision=lax.Precision.HIGHEST) + bls_ref[...]


def _out_call(raw2, hp, degp, W_mu, b_mu, W_ls, b_ls):
    return pl.pallas_call(
        _out_body,
        grid=(_GRID,),
        in_specs=[
            pl.BlockSpec((NCORES, _BLK, C), lambda i: (0, i, 0)),
            pl.BlockSpec((_BLK, C), lambda i: (i, 0)),
            pl.BlockSpec((NCORES, _BLK, 16), lambda i: (0, i, 0)),
            pl.BlockSpec((C, OC), lambda i: (0, 0)),
            pl.BlockSpec((1, OC), lambda i: (0, 0)),
            pl.BlockSpec((C, OC), lambda i: (0, 0)),
            pl.BlockSpec((1, OC), lambda i: (0, 0)),
        ],
        out_specs=[
            pl.BlockSpec((_BLK, OC), lambda i: (i, 0)),
            pl.BlockSpec((_BLK, OC), lambda i: (i, 0)),
        ],
        out_shape=[
            jax.ShapeDtypeStruct((N, OC), jnp.float32),
            jax.ShapeDtypeStruct((N, OC), jnp.float32),
        ],
    )(raw2, hp, degp, W_mu, b_mu, W_ls, b_ls)


def kernel(x, W1, b1, W_mu, b_mu, W_ls, b_ls, edge_index):
    er = edge_index.astype(jnp.int32).reshape(2, NT, NCHUNK, B)
    degp = _deg_kernel(er)
    h1 = _mm_call(x, W1)
    h1p = _scale_call(h1, degp)
    raw1 = _agg_kernel(h1p, er)
    hp = _hp_call(raw1, h1p, degp, b1.reshape(1, C))
    raw2 = _agg_kernel(hp, er)
    mu, ls = _out_call(raw2, hp, degp, W_mu, b_mu.reshape(1, OC),
                       W_ls, b_ls.reshape(1, OC))
    return (mu, ls)


# speedup vs baseline: 2.0847x; 1.0994x over previous
import functools

import jax
import jax.numpy as jnp
from jax import lax
from jax.experimental import pallas as pl
from jax.experimental.pallas import tpu as pltpu
from jax.experimental.pallas import tpu_sc as plsc

N = 10000
E = 320000
C = 64
OC = 32
NCORES = 2
NSUB = 16
NT = NCORES * NSUB
EPT = E // NT
NPAD = 10240
B = 125
NCHUNK = EPT // B
RPT = NPAD // NSUB
ZBLK = 128

_mesh = plsc.VectorSubcoreMesh(
    core_axis_name="c", subcore_axis_name="s",
    num_cores=NCORES, num_subcores=NSUB)


def _fill_f32(ref, rows, cols, value):
    def body(i, carry):
        for j in range(cols // 16):
            ref[i, pl.ds(j * 16, 16)] = jnp.full((16,), value, jnp.float32)
        return carry
    lax.fori_loop(0, rows, body, 0)


def _deg_body(er_hbm, out_hbm, dst_v, ones_v, zero_v, acc_sh):
    cid = lax.axis_index("c")
    sid = lax.axis_index("s")
    wid = cid * NSUB + sid
    _fill_f32(ones_v, B, 16, 1.0)
    _fill_f32(zero_v, ZBLK, 16, 0.0)
    for k in range(RPT // ZBLK):
        pltpu.sync_copy(zero_v, acc_sh.at[pl.ds(sid * RPT + k * ZBLK, ZBLK)])
    pltpu.sync_copy(er_hbm.at[1, wid], dst_v)
    plsc.subcore_barrier()

    def body(ci, carry):
        pltpu.sync_copy(ones_v, acc_sh.at[dst_v.at[ci]], add=True)
        return carry
    lax.fori_loop(0, NCHUNK, body, 0)

    plsc.subcore_barrier()
    pltpu.sync_copy(acc_sh.at[pl.ds(sid * RPT, RPT)],
                    out_hbm.at[cid, pl.ds(sid * RPT, RPT)])


def _make_deg_kernel(interpret=False):
    return functools.partial(
        pl.kernel,
        out_type=jax.ShapeDtypeStruct((NCORES, NPAD, 16), jnp.float32),
        mesh=_mesh,
        scratch_types=[
            pltpu.VMEM((NCHUNK, B), jnp.int32),
            pltpu.VMEM((B, 16), jnp.float32),
            pltpu.VMEM((ZBLK, 16), jnp.float32),
            pltpu.VMEM_SHARED((NPAD, 16), jnp.float32),
        ],
        compiler_params=pltpu.CompilerParams(use_tc_tiling_on_sc=False),
        interpret=interpret,
    )(_deg_body)


_deg_kernel = _make_deg_kernel()


def _agg_body(h_hbm, er_hbm, out_hbm, src_v, dst_v, rows0_v, rows1_v, zero_v,
              acc_sh, sem0, sem1):
    cid = lax.axis_index("c")
    sid = lax.axis_index("s")
    wid = cid * NSUB + sid
    _fill_f32(zero_v, ZBLK, C, 0.0)
    for k in range(RPT // ZBLK):
        pltpu.sync_copy(zero_v, acc_sh.at[pl.ds(sid * RPT + k * ZBLK, ZBLK)])
    pltpu.sync_copy(er_hbm.at[0, wid], src_v)
    pltpu.sync_copy(er_hbm.at[1, wid], dst_v)
    plsc.subcore_barrier()

    pltpu.async_copy(h_hbm.at[src_v.at[0]], rows0_v, sem0)

    @pl.loop(0, NCHUNK - 2, step=2)
    def _pair(ci):
        pltpu.async_copy(h_hbm.at[src_v.at[ci + 1]], rows1_v, sem1)
        pltpu.make_async_copy(h_hbm.at[src_v.at[ci]], rows0_v, sem0).wait()
        pltpu.sync_copy(rows0_v, acc_sh.at[dst_v.at[ci]], add=True)
        pltpu.async_copy(h_hbm.at[src_v.at[ci + 2]], rows0_v, sem0)
        pltpu.make_async_copy(h_hbm.at[src_v.at[ci + 1]], rows1_v, sem1).wait()
        pltpu.sync_copy(rows1_v, acc_sh.at[dst_v.at[ci + 1]], add=True)

    pltpu.async_copy(h_hbm.at[src_v.at[NCHUNK - 1]], rows1_v, sem1)
    pltpu.make_async_copy(h_hbm.at[src_v.at[NCHUNK - 2]], rows0_v, sem0).wait()
    pltpu.sync_copy(rows0_v, acc_sh.at[dst_v.at[NCHUNK - 2]], add=True)
    pltpu.make_async_copy(h_hbm.at[src_v.at[NCHUNK - 1]], rows1_v, sem1).wait()
    pltpu.sync_copy(rows1_v, acc_sh.at[dst_v.at[NCHUNK - 1]], add=True)

    plsc.subcore_barrier()
    pltpu.sync_copy(acc_sh.at[pl.ds(sid * RPT, RPT)],
                    out_hbm.at[cid, pl.ds(sid * RPT, RPT)])


def _make_agg_kernel(interpret=False):
    return functools.partial(
        pl.kernel,
        out_type=jax.ShapeDtypeStruct((NCORES, NPAD, C), jnp.float32),
        mesh=_mesh,
        scratch_types=[
            pltpu.VMEM((NCHUNK, B), jnp.int32),
            pltpu.VMEM((NCHUNK, B), jnp.int32),
            pltpu.VMEM((B, C), jnp.float32),
            pltpu.VMEM((B, C), jnp.float32),
            pltpu.VMEM((ZBLK, C), jnp.float32),
            pltpu.VMEM_SHARED((NPAD, C), jnp.float32),
            pltpu.SemaphoreType.DMA,
            pltpu.SemaphoreType.DMA,
        ],
        compiler_params=pltpu.CompilerParams(use_tc_tiling_on_sc=False),
        interpret=interpret,
    )(_agg_body)


_agg_kernel = _make_agg_kernel()


_BLK = 1000
_GRID = N // _BLK


def _dinv_block(degp):
    deg = degp[0, :, :1] + degp[1, :, :1] + 1.0
    return lax.rsqrt(deg)


def _mm_body(x_ref, w_ref, o_ref):
    o_ref[...] = jnp.dot(x_ref[...], w_ref[...],
                         preferred_element_type=jnp.float32,
                         precision=lax.Precision.HIGHEST)


def _mm_call(x, W1):
    return pl.pallas_call(
        _mm_body,
        grid=(_GRID,),
        in_specs=[
            pl.BlockSpec((_BLK, 128), lambda i: (i, 0)),
            pl.BlockSpec((128, C), lambda i: (0, 0)),
        ],
        out_specs=pl.BlockSpec((_BLK, C), lambda i: (i, 0)),
        out_shape=jax.ShapeDtypeStruct((N, C), jnp.float32),
    )(x, W1)


def _scale_body(h_ref, degp_ref, o_ref):
    o_ref[...] = h_ref[...] * _dinv_block(degp_ref[...])


def _scale_call(h1, degp):
    return pl.pallas_call(
        _scale_body,
        grid=(_GRID,),
        in_specs=[
            pl.BlockSpec((_BLK, C), lambda i: (i, 0)),
            pl.BlockSpec((NCORES, _BLK, 16), lambda i: (0, i, 0)),
        ],
        out_specs=pl.BlockSpec((_BLK, C), lambda i: (i, 0)),
        out_shape=jax.ShapeDtypeStruct((N, C), jnp.float32),
    )(h1, degp)


def _hp_body(raw_ref, h1p_ref, degp_ref, b_ref, o_ref):
    dinv = _dinv_block(degp_ref[...])
    raw = raw_ref[0] + raw_ref[1]
    h = jnp.maximum(dinv * (raw + h1p_ref[...]) + b_ref[...], 0.0)
    o_ref[...] = dinv * h


def _hp_call(raw1, h1p, degp, b1):
    return pl.pallas_call(
        _hp_body,
        grid=(_GRID,),
        in_specs=[
            pl.BlockSpec((NCORES, _BLK, C), lambda i: (0, i, 0)),
            pl.BlockSpec((_BLK, C), lambda i: (i, 0)),
            pl.BlockSpec((NCORES, _BLK, 16), lambda i: (0, i, 0)),
            pl.BlockSpec((1, C), lambda i: (0, 0)),
        ],
        out_specs=pl.BlockSpec((_BLK, C), lambda i: (i, 0)),
        out_shape=jax.ShapeDtypeStruct((N, C), jnp.float32),
    )(raw1, h1p, degp, b1)


def _out_body(raw_ref, hp_ref, degp_ref, wmu_ref, bmu_ref, wls_ref, bls_ref,
              mu_ref, ls_ref):
    dinv = _dinv_block(degp_ref[...])
    g = dinv * (raw_ref[0] + raw_ref[1] + hp_ref[...])
    mu_ref[...] = jnp.dot(g, wmu_ref[...],
                          preferred_element_type=jnp.float32,
                          precision=lax.Precision.HIGHEST) + bmu_ref[...]
    ls_ref[...] = jnp.dot(g, wls_ref[...],
                          preferred_element_type=jnp.float32,
                          precision=lax.Precision.HIGHEST) + bls_ref[...]


def _out_call(raw2, hp, degp, W_mu, b_mu, W_ls, b_ls):
    return pl.pallas_call(
        _out_body,
        grid=(_GRID,),
        in_specs=[
            pl.BlockSpec((NCORES, _BLK, C), lambda i: (0, i, 0)),
            pl.BlockSpec((_BLK, C), lambda i: (i, 0)),
            pl.BlockSpec((NCORES, _BLK, 16), lambda i: (0, i, 0)),
            pl.BlockSpec((C, OC), lambda i: (0, 0)),
            pl.BlockSpec((1, OC), lambda i: (0, 0)),
            pl.BlockSpec((C, OC), lambda i: (0, 0)),
            pl.BlockSpec((1, OC), lambda i: (0, 0)),
        ],
        out_specs=[
            pl.BlockSpec((_BLK, OC), lambda i: (i, 0)),
            pl.BlockSpec((_BLK, OC), lambda i: (i, 0)),
        ],
        out_shape=[
            jax.ShapeDtypeStruct((N, OC), jnp.float32),
            jax.ShapeDtypeStruct((N, OC), jnp.float32),
        ],
    )(raw2, hp, degp, W_mu, b_mu, W_ls, b_ls)


def kernel(x, W1, b1, W_mu, b_mu, W_ls, b_ls, edge_index):
    er = edge_index.astype(jnp.int32).reshape(2, NT, NCHUNK, B)
    degp = _deg_kernel(er)
    h1 = _mm_call(x, W1)
    h1p = _scale_call(h1, degp)
    raw1 = _agg_kernel(h1p, er)
    hp = _hp_call(raw1, h1p, degp, b1.reshape(1, C))
    raw2 = _agg_kernel(hp, er)
    mu, ls = _out_call(raw2, hp, degp, W_mu, b_mu.reshape(1, OC),
                       W_ls, b_ls.reshape(1, OC))
    return (mu, ls)


# speedup vs baseline: 2.1206x; 1.0172x over previous
import functools

import jax
import jax.numpy as jnp
from jax import lax
from jax.experimental import pallas as pl
from jax.experimental.pallas import tpu as pltpu
from jax.experimental.pallas import tpu_sc as plsc

N = 10000
E = 320000
C = 64
OC = 32
NCORES = 2
NSUB = 16
NT = NCORES * NSUB
EPT = E // NT
NPAD = 10240
B = 125
NCHUNK = EPT // B
RPT = NPAD // NSUB
ZBLK = 128

_mesh = plsc.VectorSubcoreMesh(
    core_axis_name="c", subcore_axis_name="s",
    num_cores=NCORES, num_subcores=NSUB)


def _fill_f32(ref, rows, cols, value):
    def body(i, carry):
        for j in range(cols // 16):
            ref[i, pl.ds(j * 16, 16)] = jnp.full((16,), value, jnp.float32)
        return carry
    lax.fori_loop(0, rows, body, 0)


def _deg_body(er_hbm, out_hbm, dst_v, ones_v, zero_v, acc_sh):
    cid = lax.axis_index("c")
    sid = lax.axis_index("s")
    wid = cid * NSUB + sid
    _fill_f32(ones_v, B, 16, 1.0)
    _fill_f32(zero_v, ZBLK, 16, 0.0)
    for k in range(RPT // ZBLK):
        pltpu.sync_copy(zero_v, acc_sh.at[pl.ds(sid * RPT + k * ZBLK, ZBLK)])
    pltpu.sync_copy(er_hbm.at[1, wid], dst_v)
    plsc.subcore_barrier()

    def body(ci, carry):
        pltpu.sync_copy(ones_v, acc_sh.at[dst_v.at[ci]], add=True)
        return carry
    lax.fori_loop(0, NCHUNK, body, 0)

    plsc.subcore_barrier()
    pltpu.sync_copy(acc_sh.at[pl.ds(sid * RPT, RPT)],
                    out_hbm.at[cid, pl.ds(sid * RPT, RPT)])


def _make_deg_kernel(interpret=False):
    return functools.partial(
        pl.kernel,
        out_type=jax.ShapeDtypeStruct((NCORES, NPAD, 16), jnp.float32),
        mesh=_mesh,
        scratch_types=[
            pltpu.VMEM((NCHUNK, B), jnp.int32),
            pltpu.VMEM((B, 16), jnp.float32),
            pltpu.VMEM((ZBLK, 16), jnp.float32),
            pltpu.VMEM_SHARED((NPAD, 16), jnp.float32),
        ],
        compiler_params=pltpu.CompilerParams(use_tc_tiling_on_sc=False),
        interpret=interpret,
    )(_deg_body)


_deg_kernel = _make_deg_kernel()


def _agg_body(h_hbm, er_hbm, out_hbm, src_v, dst_v, rows0_v, rows1_v, zero_v,
              acc_sh, sem0, sem1):
    cid = lax.axis_index("c")
    sid = lax.axis_index("s")
    wid = cid * NSUB + sid
    _fill_f32(zero_v, ZBLK, C, 0.0)
    for k in range(RPT // ZBLK):
        pltpu.sync_copy(zero_v, acc_sh.at[pl.ds(sid * RPT + k * ZBLK, ZBLK)])
    pltpu.sync_copy(er_hbm.at[0, wid], src_v)
    pltpu.sync_copy(er_hbm.at[1, wid], dst_v)
    plsc.subcore_barrier()

    pltpu.async_copy(h_hbm.at[src_v.at[0]], rows0_v, sem0)

    @pl.loop(0, NCHUNK - 2, step=2)
    def _pair(ci):
        pltpu.async_copy(h_hbm.at[src_v.at[ci + 1]], rows1_v, sem1)
        pltpu.make_async_copy(h_hbm.at[src_v.at[ci]], rows0_v, sem0).wait()
        pltpu.sync_copy(rows0_v, acc_sh.at[dst_v.at[ci]], add=True)
        pltpu.async_copy(h_hbm.at[src_v.at[ci + 2]], rows0_v, sem0)
        pltpu.make_async_copy(h_hbm.at[src_v.at[ci + 1]], rows1_v, sem1).wait()
        pltpu.sync_copy(rows1_v, acc_sh.at[dst_v.at[ci + 1]], add=True)

    pltpu.async_copy(h_hbm.at[src_v.at[NCHUNK - 1]], rows1_v, sem1)
    pltpu.make_async_copy(h_hbm.at[src_v.at[NCHUNK - 2]], rows0_v, sem0).wait()
    pltpu.sync_copy(rows0_v, acc_sh.at[dst_v.at[NCHUNK - 2]], add=True)
    pltpu.make_async_copy(h_hbm.at[src_v.at[NCHUNK - 1]], rows1_v, sem1).wait()
    pltpu.sync_copy(rows1_v, acc_sh.at[dst_v.at[NCHUNK - 1]], add=True)

    plsc.subcore_barrier()
    pltpu.sync_copy(acc_sh.at[pl.ds(sid * RPT, RPT)],
                    out_hbm.at[cid, pl.ds(sid * RPT, RPT)])


def _make_agg_kernel(interpret=False):
    return functools.partial(
        pl.kernel,
        out_type=jax.ShapeDtypeStruct((NCORES, NPAD, C), jnp.float32),
        mesh=_mesh,
        scratch_types=[
            pltpu.VMEM((NCHUNK, B), jnp.int32),
            pltpu.VMEM((NCHUNK, B), jnp.int32),
            pltpu.VMEM((B, C), jnp.float32),
            pltpu.VMEM((B, C), jnp.float32),
            pltpu.VMEM((ZBLK, C), jnp.float32),
            pltpu.VMEM_SHARED((NPAD, C), jnp.float32),
            pltpu.SemaphoreType.DMA,
            pltpu.SemaphoreType.DMA,
        ],
        compiler_params=pltpu.CompilerParams(use_tc_tiling_on_sc=False),
        interpret=interpret,
    )(_agg_body)


_agg_kernel = _make_agg_kernel()


_BLK = 10000
_GRID = N // _BLK


def _dinv_block(degp):
    deg = degp[0, :, :1] + degp[1, :, :1] + 1.0
    return lax.rsqrt(deg)


def _mm_body(x_ref, w_ref, o_ref):
    o_ref[...] = jnp.dot(x_ref[...], w_ref[...],
                         preferred_element_type=jnp.float32,
                         precision=lax.Precision.HIGHEST)


def _mm_call(x, W1):
    return pl.pallas_call(
        _mm_body,
        grid=(_GRID,),
        in_specs=[
            pl.BlockSpec((_BLK, 128), lambda i: (i, 0)),
            pl.BlockSpec((128, C), lambda i: (0, 0)),
        ],
        out_specs=pl.BlockSpec((_BLK, C), lambda i: (i, 0)),
        out_shape=jax.ShapeDtypeStruct((N, C), jnp.float32),
    )(x, W1)


def _scale_body(h_ref, degp_ref, o_ref):
    o_ref[...] = h_ref[...] * _dinv_block(degp_ref[...])


def _scale_call(h1, degp):
    return pl.pallas_call(
        _scale_body,
        grid=(_GRID,),
        in_specs=[
            pl.BlockSpec((_BLK, C), lambda i: (i, 0)),
            pl.BlockSpec((NCORES, _BLK, 16), lambda i: (0, i, 0)),
        ],
        out_specs=pl.BlockSpec((_BLK, C), lambda i: (i, 0)),
        out_shape=jax.ShapeDtypeStruct((N, C), jnp.float32),
    )(h1, degp)


def _hp_body(raw_ref, h1p_ref, degp_ref, b_ref, o_ref):
    dinv = _dinv_block(degp_ref[...])
    raw = raw_ref[0] + raw_ref[1]
    h = jnp.maximum(dinv * (raw + h1p_ref[...]) + b_ref[...], 0.0)
    o_ref[...] = dinv * h


def _hp_call(raw1, h1p, degp, b1):
    return pl.pallas_call(
        _hp_body,
        grid=(_GRID,),
        in_specs=[
            pl.BlockSpec((NCORES, _BLK, C), lambda i: (0, i, 0)),
            pl.BlockSpec((_BLK, C), lambda i: (i, 0)),
            pl.BlockSpec((NCORES, _BLK, 16), lambda i: (0, i, 0)),
            pl.BlockSpec((1, C), lambda i: (0, 0)),
        ],
        out_specs=pl.BlockSpec((_BLK, C), lambda i: (i, 0)),
        out_shape=jax.ShapeDtypeStruct((N, C), jnp.float32),
    )(raw1, h1p, degp, b1)


def _out_body(raw_ref, hp_ref, degp_ref, wmu_ref, bmu_ref, wls_ref, bls_ref,
              mu_ref, ls_ref):
    dinv = _dinv_block(degp_ref[...])
    g = dinv * (raw_ref[0] + raw_ref[1] + hp_ref[...])
    mu_ref[...] = jnp.dot(g, wmu_ref[...],
                          preferred_element_type=jnp.float32,
                          precision=lax.Precision.HIGHEST) + bmu_ref[...]
    ls_ref[...] = jnp.dot(g, wls_ref[...],
                          preferred_element_type=jnp.float32,
                          precision=lax.Precision.HIGHEST) + bls_ref[...]


def _out_call(raw2, hp, degp, W_mu, b_mu, W_ls, b_ls):
    return pl.pallas_call(
        _out_body,
        grid=(_GRID,),
        in_specs=[
            pl.BlockSpec((NCORES, _BLK, C), lambda i: (0, i, 0)),
            pl.BlockSpec((_BLK, C), lambda i: (i, 0)),
            pl.BlockSpec((NCORES, _BLK, 16), lambda i: (0, i, 0)),
            pl.BlockSpec((C, OC), lambda i: (0, 0)),
            pl.BlockSpec((1, OC), lambda i: (0, 0)),
            pl.BlockSpec((C, OC), lambda i: (0, 0)),
            pl.BlockSpec((1, OC), lambda i: (0, 0)),
        ],
        out_specs=[
            pl.BlockSpec((_BLK, OC), lambda i: (i, 0)),
            pl.BlockSpec((_BLK, OC), lambda i: (i, 0)),
        ],
        out_shape=[
            jax.ShapeDtypeStruct((N, OC), jnp.float32),
            jax.ShapeDtypeStruct((N, OC), jnp.float32),
        ],
    )(raw2, hp, degp, W_mu, b_mu, W_ls, b_ls)


def kernel(x, W1, b1, W_mu, b_mu, W_ls, b_ls, edge_index):
    er = edge_index.astype(jnp.int32).reshape(2, NT, NCHUNK, B)
    degp = _deg_kernel(er)
    h1 = _mm_call(x, W1)
    h1p = _scale_call(h1, degp)
    raw1 = _agg_kernel(h1p, er)
    hp = _hp_call(raw1, h1p, degp, b1.reshape(1, C))
    raw2 = _agg_kernel(hp, er)
    mu, ls = _out_call(raw2, hp, degp, W_mu, b_mu.reshape(1, OC),
                       W_ls, b_ls.reshape(1, OC))
    return (mu, ls)


# speedup vs baseline: 2.1700x; 1.0233x over previous
import functools

import jax
import jax.numpy as jnp
from jax import lax
from jax.experimental import pallas as pl
from jax.experimental.pallas import tpu as pltpu
from jax.experimental.pallas import tpu_sc as plsc

N = 10000
E = 320000
C = 64
OC = 32
NCORES = 2
NSUB = 16
NT = NCORES * NSUB
EPT = E // NT
NPAD = 10240
B = 125
NCHUNK = EPT // B
RPT = NPAD // NSUB
ZBLK = 128

_mesh = plsc.VectorSubcoreMesh(
    core_axis_name="c", subcore_axis_name="s",
    num_cores=NCORES, num_subcores=NSUB)


def _fill_f32(ref, rows, cols, value):
    def body(i, carry):
        for j in range(cols // 16):
            ref[i, pl.ds(j * 16, 16)] = jnp.full((16,), value, jnp.float32)
        return carry
    lax.fori_loop(0, rows, body, 0)


def _deg_body(er_hbm, out_hbm, dst_v, ones_v, zero_v, acc_sh):
    cid = lax.axis_index("c")
    sid = lax.axis_index("s")
    wid = cid * NSUB + sid
    _fill_f32(ones_v, B, 16, 1.0)
    _fill_f32(zero_v, ZBLK, 16, 0.0)
    for k in range(RPT // ZBLK):
        pltpu.sync_copy(zero_v, acc_sh.at[pl.ds(sid * RPT + k * ZBLK, ZBLK)])
    pltpu.sync_copy(er_hbm.at[1, wid], dst_v)
    plsc.subcore_barrier()

    def body(ci, carry):
        pltpu.sync_copy(ones_v, acc_sh.at[dst_v.at[ci]], add=True)
        return carry
    lax.fori_loop(0, NCHUNK, body, 0)

    plsc.subcore_barrier()
    pltpu.sync_copy(acc_sh.at[pl.ds(sid * RPT, RPT)],
                    out_hbm.at[cid, pl.ds(sid * RPT, RPT)])


def _make_deg_kernel(interpret=False):
    return functools.partial(
        pl.kernel,
        out_type=jax.ShapeDtypeStruct((NCORES, NPAD, 16), jnp.float32),
        mesh=_mesh,
        scratch_types=[
            pltpu.VMEM((NCHUNK, B), jnp.int32),
            pltpu.VMEM((B, 16), jnp.float32),
            pltpu.VMEM((ZBLK, 16), jnp.float32),
            pltpu.VMEM_SHARED((NPAD, 16), jnp.float32),
        ],
        compiler_params=pltpu.CompilerParams(use_tc_tiling_on_sc=False),
        interpret=interpret,
    )(_deg_body)


_deg_kernel = _make_deg_kernel()


def _agg_body(h_hbm, er_hbm, out_hbm, src_v, dst_v, rows0_v, rows1_v, zero_v,
              acc_sh, sem0, sem1):
    cid = lax.axis_index("c")
    sid = lax.axis_index("s")
    wid = cid * NSUB + sid
    _fill_f32(zero_v, ZBLK, C, 0.0)
    for k in range(RPT // ZBLK):
        pltpu.sync_copy(zero_v, acc_sh.at[pl.ds(sid * RPT + k * ZBLK, ZBLK)])
    pltpu.sync_copy(er_hbm.at[0, wid], src_v)
    pltpu.sync_copy(er_hbm.at[1, wid], dst_v)
    plsc.subcore_barrier()

    pltpu.async_copy(h_hbm.at[src_v.at[0]], rows0_v, sem0)

    @pl.loop(0, NCHUNK - 2, step=2)
    def _pair(ci):
        pltpu.async_copy(h_hbm.at[src_v.at[ci + 1]], rows1_v, sem1)
        pltpu.make_async_copy(h_hbm.at[src_v.at[ci]], rows0_v, sem0).wait()
        pltpu.sync_copy(rows0_v, acc_sh.at[dst_v.at[ci]], add=True)
        pltpu.async_copy(h_hbm.at[src_v.at[ci + 2]], rows0_v, sem0)
        pltpu.make_async_copy(h_hbm.at[src_v.at[ci + 1]], rows1_v, sem1).wait()
        pltpu.sync_copy(rows1_v, acc_sh.at[dst_v.at[ci + 1]], add=True)

    pltpu.async_copy(h_hbm.at[src_v.at[NCHUNK - 1]], rows1_v, sem1)
    pltpu.make_async_copy(h_hbm.at[src_v.at[NCHUNK - 2]], rows0_v, sem0).wait()
    pltpu.sync_copy(rows0_v, acc_sh.at[dst_v.at[NCHUNK - 2]], add=True)
    pltpu.make_async_copy(h_hbm.at[src_v.at[NCHUNK - 1]], rows1_v, sem1).wait()
    pltpu.sync_copy(rows1_v, acc_sh.at[dst_v.at[NCHUNK - 1]], add=True)

    plsc.subcore_barrier()
    pltpu.sync_copy(acc_sh.at[pl.ds(sid * RPT, RPT)],
                    out_hbm.at[cid, pl.ds(sid * RPT, RPT)])


def _make_agg_kernel(interpret=False):
    return functools.partial(
        pl.kernel,
        out_type=jax.ShapeDtypeStruct((NCORES, NPAD, C), jnp.float32),
        mesh=_mesh,
        scratch_types=[
            pltpu.VMEM((NCHUNK, B), jnp.int32),
            pltpu.VMEM((NCHUNK, B), jnp.int32),
            pltpu.VMEM((B, C), jnp.float32),
            pltpu.VMEM((B, C), jnp.float32),
            pltpu.VMEM((ZBLK, C), jnp.float32),
            pltpu.VMEM_SHARED((NPAD, C), jnp.float32),
            pltpu.SemaphoreType.DMA,
            pltpu.SemaphoreType.DMA,
        ],
        compiler_params=pltpu.CompilerParams(use_tc_tiling_on_sc=False),
        interpret=interpret,
    )(_agg_body)


_agg_kernel = _make_agg_kernel()


_BLK = 2000
_GRID = N // _BLK


def _dinv_block(degp):
    deg = degp[0, :, :1] + degp[1, :, :1] + 1.0
    return lax.rsqrt(deg)


def _mm_body(x_ref, w_ref, o_ref):
    o_ref[...] = jnp.dot(x_ref[...], w_ref[...],
                         preferred_element_type=jnp.float32,
                         precision=lax.Precision.HIGHEST)


def _mm_call(x, W1):
    return pl.pallas_call(
        _mm_body,
        grid=(_GRID,),
        in_specs=[
            pl.BlockSpec((_BLK, 128), lambda i: (i, 0)),
            pl.BlockSpec((128, C), lambda i: (0, 0)),
        ],
        out_specs=pl.BlockSpec((_BLK, C), lambda i: (i, 0)),
        out_shape=jax.ShapeDtypeStruct((N, C), jnp.float32),
    )(x, W1)


def _scale_body(h_ref, degp_ref, o_ref):
    o_ref[...] = h_ref[...] * _dinv_block(degp_ref[...])


def _scale_call(h1, degp):
    return pl.pallas_call(
        _scale_body,
        grid=(_GRID,),
        in_specs=[
            pl.BlockSpec((_BLK, C), lambda i: (i, 0)),
            pl.BlockSpec((NCORES, _BLK, 16), lambda i: (0, i, 0)),
        ],
        out_specs=pl.BlockSpec((_BLK, C), lambda i: (i, 0)),
        out_shape=jax.ShapeDtypeStruct((N, C), jnp.float32),
    )(h1, degp)


def _hp_body(raw_ref, h1p_ref, degp_ref, b_ref, o_ref):
    dinv = _dinv_block(degp_ref[...])
    raw = raw_ref[0] + raw_ref[1]
    h = jnp.maximum(dinv * (raw + h1p_ref[...]) + b_ref[...], 0.0)
    o_ref[...] = dinv * h


def _hp_call(raw1, h1p, degp, b1):
    return pl.pallas_call(
        _hp_body,
        grid=(_GRID,),
        in_specs=[
            pl.BlockSpec((NCORES, _BLK, C), lambda i: (0, i, 0)),
            pl.BlockSpec((_BLK, C), lambda i: (i, 0)),
            pl.BlockSpec((NCORES, _BLK, 16), lambda i: (0, i, 0)),
            pl.BlockSpec((1, C), lambda i: (0, 0)),
        ],
        out_specs=pl.BlockSpec((_BLK, C), lambda i: (i, 0)),
        out_shape=jax.ShapeDtypeStruct((N, C), jnp.float32),
    )(raw1, h1p, degp, b1)


def _out_body(raw_ref, hp_ref, degp_ref, wmu_ref, bmu_ref, wls_ref, bls_ref,
              mu_ref, ls_ref):
    dinv = _dinv_block(degp_ref[...])
    g = dinv * (raw_ref[0] + raw_ref[1] + hp_ref[...])
    mu_ref[...] = jnp.dot(g, wmu_ref[...],
                          preferred_element_type=jnp.float32,
                          precision=lax.Precision.HIGHEST) + bmu_ref[...]
    ls_ref[...] = jnp.dot(g, wls_ref[...],
                          preferred_element_type=jnp.float32,
                          precision=lax.Precision.HIGHEST) + bls_ref[...]


def _out_call(raw2, hp, degp, W_mu, b_mu, W_ls, b_ls):
    return pl.pallas_call(
        _out_body,
        grid=(_GRID,),
        in_specs=[
            pl.BlockSpec((NCORES, _BLK, C), lambda i: (0, i, 0)),
            pl.BlockSpec((_BLK, C), lambda i: (i, 0)),
            pl.BlockSpec((NCORES, _BLK, 16), lambda i: (0, i, 0)),
            pl.BlockSpec((C, OC), lambda i: (0, 0)),
            pl.BlockSpec((1, OC), lambda i: (0, 0)),
            pl.BlockSpec((C, OC), lambda i: (0, 0)),
            pl.BlockSpec((1, OC), lambda i: (0, 0)),
        ],
        out_specs=[
            pl.BlockSpec((_BLK, OC), lambda i: (i, 0)),
            pl.BlockSpec((_BLK, OC), lambda i: (i, 0)),
        ],
        out_shape=[
            jax.ShapeDtypeStruct((N, OC), jnp.float32),
            jax.ShapeDtypeStruct((N, OC), jnp.float32),
        ],
    )(raw2, hp, degp, W_mu, b_mu, W_ls, b_ls)


def kernel(x, W1, b1, W_mu, b_mu, W_ls, b_ls, edge_index):
    er = edge_index.astype(jnp.int32).reshape(2, NT, NCHUNK, B)
    degp = _deg_kernel(er)
    h1 = _mm_call(x, W1)
    h1p = _scale_call(h1, degp)
    raw1 = _agg_kernel(h1p, er)
    hp = _hp_call(raw1, h1p, degp, b1.reshape(1, C))
    raw2 = _agg_kernel(hp, er)
    mu, ls = _out_call(raw2, hp, degp, W_mu, b_mu.reshape(1, OC),
                       W_ls, b_ls.reshape(1, OC))
    return (mu, ls)


# speedup vs baseline: 2.3460x; 1.0811x over previous
import functools

import jax
import jax.numpy as jnp
from jax import lax
from jax.experimental import pallas as pl
from jax.experimental.pallas import tpu as pltpu
from jax.experimental.pallas import tpu_sc as plsc

N = 10000
E = 320000
C = 64
OC = 32
NCORES = 2
NSUB = 16
NT = NCORES * NSUB
EPT = E // NT
NPAD = 10240
B = 125
NCHUNK = EPT // B
RPT = NPAD // NSUB
ZBLK = 128

_mesh = plsc.VectorSubcoreMesh(
    core_axis_name="c", subcore_axis_name="s",
    num_cores=NCORES, num_subcores=NSUB)


def _fill_f32(ref, rows, cols, value):
    def body(i, carry):
        for j in range(cols // 16):
            ref[i, pl.ds(j * 16, 16)] = jnp.full((16,), value, jnp.float32)
        return carry
    lax.fori_loop(0, rows, body, 0)


def _deg_body(er_hbm, out_hbm, dst_v, ones_v, zero_v, acc_sh):
    cid = lax.axis_index("c")
    sid = lax.axis_index("s")
    wid = cid * NSUB + sid
    _fill_f32(ones_v, B, 16, 1.0)
    _fill_f32(zero_v, ZBLK, 16, 0.0)
    for k in range(RPT // ZBLK):
        pltpu.sync_copy(zero_v, acc_sh.at[pl.ds(sid * RPT + k * ZBLK, ZBLK)])
    pltpu.sync_copy(er_hbm.at[1, wid], dst_v)
    plsc.subcore_barrier()

    def body(ci, carry):
        pltpu.sync_copy(ones_v, acc_sh.at[dst_v.at[ci]], add=True)
        return carry
    lax.fori_loop(0, NCHUNK, body, 0)

    plsc.subcore_barrier()
    pltpu.sync_copy(acc_sh.at[pl.ds(sid * RPT, RPT)],
                    out_hbm.at[cid, pl.ds(sid * RPT, RPT)])


def _make_deg_kernel(interpret=False):
    return functools.partial(
        pl.kernel,
        out_type=jax.ShapeDtypeStruct((NCORES, NPAD, 16), jnp.float32),
        mesh=_mesh,
        scratch_types=[
            pltpu.VMEM((NCHUNK, B), jnp.int32),
            pltpu.VMEM((B, 16), jnp.float32),
            pltpu.VMEM((ZBLK, 16), jnp.float32),
            pltpu.VMEM_SHARED((NPAD, 16), jnp.float32),
        ],
        compiler_params=pltpu.CompilerParams(use_tc_tiling_on_sc=False),
        interpret=interpret,
    )(_deg_body)


_deg_kernel = _make_deg_kernel()


def _agg_body(h_hbm, er_hbm, out_hbm, src_v, dst_v, rows0_v, rows1_v, zero_v,
              acc_sh, sem0, sem1):
    cid = lax.axis_index("c")
    sid = lax.axis_index("s")
    wid = cid * NSUB + sid
    _fill_f32(zero_v, ZBLK, C, 0.0)
    for k in range(RPT // ZBLK):
        pltpu.sync_copy(zero_v, acc_sh.at[pl.ds(sid * RPT + k * ZBLK, ZBLK)])
    pltpu.sync_copy(er_hbm.at[0, wid], src_v)
    pltpu.sync_copy(er_hbm.at[1, wid], dst_v)
    plsc.subcore_barrier()

    pltpu.async_copy(h_hbm.at[src_v.at[0]], rows0_v, sem0)

    @pl.loop(0, NCHUNK - 2, step=2)
    def _pair(ci):
        pltpu.async_copy(h_hbm.at[src_v.at[ci + 1]], rows1_v, sem1)
        pltpu.make_async_copy(h_hbm.at[src_v.at[ci]], rows0_v, sem0).wait()
        pltpu.sync_copy(rows0_v, acc_sh.at[dst_v.at[ci]], add=True)
        pltpu.async_copy(h_hbm.at[src_v.at[ci + 2]], rows0_v, sem0)
        pltpu.make_async_copy(h_hbm.at[src_v.at[ci + 1]], rows1_v, sem1).wait()
        pltpu.sync_copy(rows1_v, acc_sh.at[dst_v.at[ci + 1]], add=True)

    pltpu.async_copy(h_hbm.at[src_v.at[NCHUNK - 1]], rows1_v, sem1)
    pltpu.make_async_copy(h_hbm.at[src_v.at[NCHUNK - 2]], rows0_v, sem0).wait()
    pltpu.sync_copy(rows0_v, acc_sh.at[dst_v.at[NCHUNK - 2]], add=True)
    pltpu.make_async_copy(h_hbm.at[src_v.at[NCHUNK - 1]], rows1_v, sem1).wait()
    pltpu.sync_copy(rows1_v, acc_sh.at[dst_v.at[NCHUNK - 1]], add=True)

    plsc.subcore_barrier()
    pltpu.sync_copy(acc_sh.at[pl.ds(sid * RPT, RPT)],
                    out_hbm.at[pl.ds(sid * RPT, RPT), pl.ds(cid * C, C)])


def _make_agg_kernel(interpret=False):
    return functools.partial(
        pl.kernel,
        out_type=jax.ShapeDtypeStruct((NPAD, NCORES * C), jnp.float32),
        mesh=_mesh,
        scratch_types=[
            pltpu.VMEM((NCHUNK, B), jnp.int32),
            pltpu.VMEM((NCHUNK, B), jnp.int32),
            pltpu.VMEM((B, C), jnp.float32),
            pltpu.VMEM((B, C), jnp.float32),
            pltpu.VMEM((ZBLK, C), jnp.float32),
            pltpu.VMEM_SHARED((NPAD, C), jnp.float32),
            pltpu.SemaphoreType.DMA,
            pltpu.SemaphoreType.DMA,
        ],
        compiler_params=pltpu.CompilerParams(use_tc_tiling_on_sc=False),
        interpret=interpret,
    )(_agg_body)


_agg_kernel = _make_agg_kernel()


_BLK = 2000
_GRID = N // _BLK


def _dinv_block(degp):
    deg = degp[0, :, :1] + degp[1, :, :1] + 1.0
    return lax.rsqrt(deg)


def _mm_body(x_ref, w_ref, o_ref):
    o_ref[...] = jnp.dot(x_ref[...], w_ref[...],
                         preferred_element_type=jnp.float32,
                         precision=lax.Precision.HIGHEST)


def _mm_call(x, W1):
    return pl.pallas_call(
        _mm_body,
        grid=(_GRID,),
        in_specs=[
            pl.BlockSpec((_BLK, 128), lambda i: (i, 0)),
            pl.BlockSpec((128, C), lambda i: (0, 0)),
        ],
        out_specs=pl.BlockSpec((_BLK, C), lambda i: (i, 0)),
        out_shape=jax.ShapeDtypeStruct((N, C), jnp.float32),
    )(x, W1)


def _scale_body(h_ref, degp_ref, o_ref):
    o_ref[...] = h_ref[...] * _dinv_block(degp_ref[...])


def _scale_call(h1, degp):
    return pl.pallas_call(
        _scale_body,
        grid=(_GRID,),
        in_specs=[
            pl.BlockSpec((_BLK, C), lambda i: (i, 0)),
            pl.BlockSpec((NCORES, _BLK, 16), lambda i: (0, i, 0)),
        ],
        out_specs=pl.BlockSpec((_BLK, C), lambda i: (i, 0)),
        out_shape=jax.ShapeDtypeStruct((N, C), jnp.float32),
    )(h1, degp)


def _hp_body(raw_ref, h1p_ref, degp_ref, b_ref, o_ref):
    dinv = _dinv_block(degp_ref[...])
    r = raw_ref[...]
    raw = r[:, :C] + r[:, C:]
    h = jnp.maximum(dinv * (raw + h1p_ref[...]) + b_ref[...], 0.0)
    o_ref[...] = dinv * h


def _hp_call(raw1, h1p, degp, b1):
    return pl.pallas_call(
        _hp_body,
        grid=(_GRID,),
        in_specs=[
            pl.BlockSpec((_BLK, NCORES * C), lambda i: (i, 0)),
            pl.BlockSpec((_BLK, C), lambda i: (i, 0)),
            pl.BlockSpec((NCORES, _BLK, 16), lambda i: (0, i, 0)),
            pl.BlockSpec((1, C), lambda i: (0, 0)),
        ],
        out_specs=pl.BlockSpec((_BLK, C), lambda i: (i, 0)),
        out_shape=jax.ShapeDtypeStruct((N, C), jnp.float32),
    )(raw1, h1p, degp, b1)


def _out_body(raw_ref, hp_ref, degp_ref, wmu_ref, bmu_ref, wls_ref, bls_ref,
              mu_ref, ls_ref):
    dinv = _dinv_block(degp_ref[...])
    r = raw_ref[...]
    g = dinv * (r[:, :C] + r[:, C:] + hp_ref[...])
    mu_ref[...] = jnp.dot(g, wmu_ref[...],
                          preferred_element_type=jnp.float32,
                          precision=lax.Precision.HIGHEST) + bmu_ref[...]
    ls_ref[...] = jnp.dot(g, wls_ref[...],
                          preferred_element_type=jnp.float32,
                          precision=lax.Precision.HIGHEST) + bls_ref[...]


def _out_call(raw2, hp, degp, W_mu, b_mu, W_ls, b_ls):
    return pl.pallas_call(
        _out_body,
        grid=(_GRID,),
        in_specs=[
            pl.BlockSpec((_BLK, NCORES * C), lambda i: (i, 0)),
            pl.BlockSpec((_BLK, C), lambda i: (i, 0)),
            pl.BlockSpec((NCORES, _BLK, 16), lambda i: (0, i, 0)),
            pl.BlockSpec((C, OC), lambda i: (0, 0)),
            pl.BlockSpec((1, OC), lambda i: (0, 0)),
            pl.BlockSpec((C, OC), lambda i: (0, 0)),
            pl.BlockSpec((1, OC), lambda i: (0, 0)),
        ],
        out_specs=[
            pl.BlockSpec((_BLK, OC), lambda i: (i, 0)),
            pl.BlockSpec((_BLK, OC), lambda i: (i, 0)),
        ],
        out_shape=[
            jax.ShapeDtypeStruct((N, OC), jnp.float32),
            jax.ShapeDtypeStruct((N, OC), jnp.float32),
        ],
    )(raw2, hp, degp, W_mu, b_mu, W_ls, b_ls)


def kernel(x, W1, b1, W_mu, b_mu, W_ls, b_ls, edge_index):
    er = edge_index.astype(jnp.int32).reshape(2, NT, NCHUNK, B)
    degp = _deg_kernel(er)
    h1 = _mm_call(x, W1)
    h1p = _scale_call(h1, degp)
    raw1 = _agg_kernel(h1p, er)
    hp = _hp_call(raw1, h1p, degp, b1.reshape(1, C))
    raw2 = _agg_kernel(hp, er)
    mu, ls = _out_call(raw2, hp, degp, W_mu, b_mu.reshape(1, OC),
                       W_ls, b_ls.reshape(1, OC))
    return (mu, ls)


# speedup vs baseline: 2.3831x; 1.0158x over previous
import functools

import jax
import jax.numpy as jnp
from jax import lax
from jax.experimental import pallas as pl
from jax.experimental.pallas import tpu as pltpu
from jax.experimental.pallas import tpu_sc as plsc

N = 10000
E = 320000
C = 64
OC = 32
NCORES = 2
NSUB = 16
NT = NCORES * NSUB
EPT = E // NT
NPAD = 10240
B = 125
NCHUNK = EPT // B
RPT = NPAD // NSUB
ZBLK = 128

_mesh = plsc.VectorSubcoreMesh(
    core_axis_name="c", subcore_axis_name="s",
    num_cores=NCORES, num_subcores=NSUB)


def _fill_f32(ref, rows, cols, value):
    def body(i, carry):
        for j in range(cols // 16):
            ref[i, pl.ds(j * 16, 16)] = jnp.full((16,), value, jnp.float32)
        return carry
    lax.fori_loop(0, rows, body, 0)


def _deg_body(er_hbm, out_hbm, dst_v, ones_v, zero_v, acc_sh, sem):
    cid = lax.axis_index("c")
    sid = lax.axis_index("s")
    wid = cid * NSUB + sid
    _fill_f32(ones_v, B, 16, 1.0)
    _fill_f32(zero_v, ZBLK, 16, 0.0)
    for k in range(RPT // ZBLK):
        pltpu.sync_copy(zero_v, acc_sh.at[pl.ds(sid * RPT + k * ZBLK, ZBLK)])
    pltpu.sync_copy(er_hbm.at[1, wid], dst_v)
    plsc.subcore_barrier()

    def body(ci, carry):
        pltpu.async_copy(ones_v, acc_sh.at[dst_v.at[ci]], sem, add=True)
        return carry
    lax.fori_loop(0, NCHUNK, body, 0)

    def drain(ci, carry):
        pltpu.make_async_copy(ones_v, acc_sh.at[dst_v.at[ci]], sem).wait()
        return carry
    lax.fori_loop(0, NCHUNK, drain, 0)

    plsc.subcore_barrier()
    pltpu.sync_copy(acc_sh.at[pl.ds(sid * RPT, RPT)],
                    out_hbm.at[pl.ds(sid * RPT, RPT), pl.ds(cid * 16, 16)])


def _make_deg_kernel(interpret=False):
    return functools.partial(
        pl.kernel,
        out_type=jax.ShapeDtypeStruct((NPAD, NCORES * 16), jnp.float32),
        mesh=_mesh,
        scratch_types=[
            pltpu.VMEM((NCHUNK, B), jnp.int32),
            pltpu.VMEM((B, 16), jnp.float32),
            pltpu.VMEM((ZBLK, 16), jnp.float32),
            pltpu.VMEM_SHARED((NPAD, 16), jnp.float32),
            pltpu.SemaphoreType.DMA,
        ],
        compiler_params=pltpu.CompilerParams(use_tc_tiling_on_sc=False),
        interpret=interpret,
    )(_deg_body)


_deg_kernel = _make_deg_kernel()


def _agg_body(h_hbm, er_hbm, out_hbm, src_v, dst_v, rows0_v, rows1_v, zero_v,
              acc_sh, sem0, sem1):
    cid = lax.axis_index("c")
    sid = lax.axis_index("s")
    wid = cid * NSUB + sid
    _fill_f32(zero_v, ZBLK, C, 0.0)
    for k in range(RPT // ZBLK):
        pltpu.sync_copy(zero_v, acc_sh.at[pl.ds(sid * RPT + k * ZBLK, ZBLK)])
    pltpu.sync_copy(er_hbm.at[0, wid], src_v)
    pltpu.sync_copy(er_hbm.at[1, wid], dst_v)
    plsc.subcore_barrier()

    pltpu.async_copy(h_hbm.at[src_v.at[0]], rows0_v, sem0)

    @pl.loop(0, NCHUNK - 2, step=2)
    def _pair(ci):
        pltpu.async_copy(h_hbm.at[src_v.at[ci + 1]], rows1_v, sem1)
        pltpu.make_async_copy(h_hbm.at[src_v.at[ci]], rows0_v, sem0).wait()
        pltpu.sync_copy(rows0_v, acc_sh.at[dst_v.at[ci]], add=True)
        pltpu.async_copy(h_hbm.at[src_v.at[ci + 2]], rows0_v, sem0)
        pltpu.make_async_copy(h_hbm.at[src_v.at[ci + 1]], rows1_v, sem1).wait()
        pltpu.sync_copy(rows1_v, acc_sh.at[dst_v.at[ci + 1]], add=True)

    pltpu.async_copy(h_hbm.at[src_v.at[NCHUNK - 1]], rows1_v, sem1)
    pltpu.make_async_copy(h_hbm.at[src_v.at[NCHUNK - 2]], rows0_v, sem0).wait()
    pltpu.sync_copy(rows0_v, acc_sh.at[dst_v.at[NCHUNK - 2]], add=True)
    pltpu.make_async_copy(h_hbm.at[src_v.at[NCHUNK - 1]], rows1_v, sem1).wait()
    pltpu.sync_copy(rows1_v, acc_sh.at[dst_v.at[NCHUNK - 1]], add=True)

    plsc.subcore_barrier()
    pltpu.sync_copy(acc_sh.at[pl.ds(sid * RPT, RPT)],
                    out_hbm.at[pl.ds(sid * RPT, RPT), pl.ds(cid * C, C)])


def _make_agg_kernel(interpret=False):
    return functools.partial(
        pl.kernel,
        out_type=jax.ShapeDtypeStruct((NPAD, NCORES * C), jnp.float32),
        mesh=_mesh,
        scratch_types=[
            pltpu.VMEM((NCHUNK, B), jnp.int32),
            pltpu.VMEM((NCHUNK, B), jnp.int32),
            pltpu.VMEM((B, C), jnp.float32),
            pltpu.VMEM((B, C), jnp.float32),
            pltpu.VMEM((ZBLK, C), jnp.float32),
            pltpu.VMEM_SHARED((NPAD, C), jnp.float32),
            pltpu.SemaphoreType.DMA,
            pltpu.SemaphoreType.DMA,
        ],
        compiler_params=pltpu.CompilerParams(use_tc_tiling_on_sc=False),
        interpret=interpret,
    )(_agg_body)


_agg_kernel = _make_agg_kernel()


_BLK = 2000
_GRID = N // _BLK


def _dinv_block(degp):
    deg = degp[:, :1] + degp[:, 16:17] + 1.0
    return lax.rsqrt(deg)


def _mm_body(x_ref, w_ref, o_ref):
    o_ref[...] = jnp.dot(x_ref[...], w_ref[...],
                         preferred_element_type=jnp.float32,
                         precision=lax.Precision.HIGHEST)


def _mm_call(x, W1):
    return pl.pallas_call(
        _mm_body,
        grid=(_GRID,),
        in_specs=[
            pl.BlockSpec((_BLK, 128), lambda i: (i, 0)),
            pl.BlockSpec((128, C), lambda i: (0, 0)),
        ],
        out_specs=pl.BlockSpec((_BLK, C), lambda i: (i, 0)),
        out_shape=jax.ShapeDtypeStruct((N, C), jnp.float32),
    )(x, W1)


def _scale_body(h_ref, degp_ref, o_ref):
    o_ref[...] = h_ref[...] * _dinv_block(degp_ref[...])


def _scale_call(h1, degp):
    return pl.pallas_call(
        _scale_body,
        grid=(_GRID,),
        in_specs=[
            pl.BlockSpec((_BLK, C), lambda i: (i, 0)),
            pl.BlockSpec((_BLK, NCORES * 16), lambda i: (i, 0)),
        ],
        out_specs=pl.BlockSpec((_BLK, C), lambda i: (i, 0)),
        out_shape=jax.ShapeDtypeStruct((N, C), jnp.float32),
    )(h1, degp)


def _hp_body(raw_ref, h1p_ref, degp_ref, b_ref, o_ref):
    dinv = _dinv_block(degp_ref[...])
    r = raw_ref[...]
    raw = r[:, :C] + r[:, C:]
    h = jnp.maximum(dinv * (raw + h1p_ref[...]) + b_ref[...], 0.0)
    o_ref[...] = dinv * h


def _hp_call(raw1, h1p, degp, b1):
    return pl.pallas_call(
        _hp_body,
        grid=(_GRID,),
        in_specs=[
            pl.BlockSpec((_BLK, NCORES * C), lambda i: (i, 0)),
            pl.BlockSpec((_BLK, C), lambda i: (i, 0)),
            pl.BlockSpec((_BLK, NCORES * 16), lambda i: (i, 0)),
            pl.BlockSpec((1, C), lambda i: (0, 0)),
        ],
        out_specs=pl.BlockSpec((_BLK, C), lambda i: (i, 0)),
        out_shape=jax.ShapeDtypeStruct((N, C), jnp.float32),
    )(raw1, h1p, degp, b1)


def _out_body(raw_ref, hp_ref, degp_ref, wmu_ref, bmu_ref, wls_ref, bls_ref,
              mu_ref, ls_ref):
    dinv = _dinv_block(degp_ref[...])
    r = raw_ref[...]
    g = dinv * (r[:, :C] + r[:, C:] + hp_ref[...])
    mu_ref[...] = jnp.dot(g, wmu_ref[...],
                          preferred_element_type=jnp.float32,
                          precision=lax.Precision.HIGHEST) + bmu_ref[...]
    ls_ref[...] = jnp.dot(g, wls_ref[...],
                          preferred_element_type=jnp.float32,
                          precision=lax.Precision.HIGHEST) + bls_ref[...]


def _out_call(raw2, hp, degp, W_mu, b_mu, W_ls, b_ls):
    return pl.pallas_call(
        _out_body,
        grid=(_GRID,),
        in_specs=[
            pl.BlockSpec((_BLK, NCORES * C), lambda i: (i, 0)),
            pl.BlockSpec((_BLK, C), lambda i: (i, 0)),
            pl.BlockSpec((_BLK, NCORES * 16), lambda i: (i, 0)),
            pl.BlockSpec((C, OC), lambda i: (0, 0)),
            pl.BlockSpec((1, OC), lambda i: (0, 0)),
            pl.BlockSpec((C, OC), lambda i: (0, 0)),
            pl.BlockSpec((1, OC), lambda i: (0, 0)),
        ],
        out_specs=[
            pl.BlockSpec((_BLK, OC), lambda i: (i, 0)),
            pl.BlockSpec((_BLK, OC), lambda i: (i, 0)),
        ],
        out_shape=[
            jax.ShapeDtypeStruct((N, OC), jnp.float32),
            jax.ShapeDtypeStruct((N, OC), jnp.float32),
        ],
    )(raw2, hp, degp, W_mu, b_mu, W_ls, b_ls)


def kernel(x, W1, b1, W_mu, b_mu, W_ls, b_ls, edge_index):
    er = edge_index.astype(jnp.int32).reshape(2, NT, NCHUNK, B)
    degp = _deg_kernel(er)
    h1 = _mm_call(x, W1)
    h1p = _scale_call(h1, degp)
    raw1 = _agg_kernel(h1p, er)
    hp = _hp_call(raw1, h1p, degp, b1.reshape(1, C))
    raw2 = _agg_kernel(hp, er)
    mu, ls = _out_call(raw2, hp, degp, W_mu, b_mu.reshape(1, OC),
                       W_ls, b_ls.reshape(1, OC))
    return (mu, ls)
